# baseline (speedup 1.0000x reference)
import functools

import jax
import jax.numpy as jnp
from jax import lax
from jax.experimental import pallas as pl
from jax.experimental.pallas import tpu as pltpu
from jax.experimental.pallas import tpu_sc as plsc

L = 16
N = 32768
B = 128
NC = 2
NS = 16
NW = NC * NS
ROWS_PER_W = B // NW
VPG = 16
GROUPS = N // (VPG * L)
NBATCH = GROUPS // 16


def _bf_max(v):
    for sh in (1, 2, 4, 8):
        v = jnp.maximum(v, v[lax.iota(jnp.int32, L) ^ sh])
    return v


def _bf_sum(v):
    for sh in (1, 2, 4, 8):
        v = v + v[lax.iota(jnp.int32, L) ^ sh]
    return v


def _bf_or(v):
    for sh in (1, 2, 4, 8):
        v = v | v[lax.iota(jnp.int32, L) ^ sh]
    return v


def _bit(g):
    return jnp.int32(-(2 ** 31)) if g == 31 else jnp.int32(1 << g)


def _sc_kernel(x_hbm, out_hbm, data_v, gmax_v, cand_v, ms_v, off_r, nq_r,
               glist_r, sem_a, sem_b):
    wid = lax.axis_index("s") * NC + lax.axis_index("c")
    zeros = jnp.zeros((L,), jnp.float32)
    izeros = jnp.zeros((L,), jnp.int32)
    ms_vec = zeros
    sems = (sem_a, sem_b)

    copies = [None, None]
    copies[0] = pltpu.async_copy(
        x_hbm.at[wid * ROWS_PER_W], data_v.at[pl.ds(0, N)], sems[0])

    for r in range(ROWS_PER_W):
        row = wid * ROWS_PER_W + r
        dbase = (r % 2) * N
        copies[r % 2].wait()
        if r + 1 < ROWS_PER_W:
            copies[(r + 1) % 2] = pltpu.async_copy(
                x_hbm.at[row + 1],
                data_v.at[pl.ds(((r + 1) % 2) * N, N)], sems[(r + 1) % 2])

        def pass_a(g2, rowvmax):
            for half in range(2):
                base = dbase + (g2 * 2 + half) * (VPG * L)
                t0_ = data_v[pl.ds(base, L)]
                t1_ = data_v[pl.ds(base + L, L)]
                for k in range(2, VPG, 2):
                    t0_ = jnp.maximum(t0_, data_v[pl.ds(base + k * L, L)])
                    t1_ = jnp.maximum(
                        t1_, data_v[pl.ds(base + (k + 1) * L, L)])
                gm = jnp.maximum(t0_, t1_)
                gmax_v[pl.ds((g2 * 2 + half) * L, L)] = gm
                rowvmax = jnp.maximum(rowvmax, gm)
            return rowvmax

        rowvmax = lax.fori_loop(
            0, GROUPS // 2, pass_a, jnp.full((L,), -1e30, jnp.float32))
        rowmax = _bf_max(rowvmax)[0]
        t0 = rowmax - 1.0

        nq_r[0] = jnp.int32(0)

        def pass_b1(b, carry):
            bits0 = izeros
            bits1 = izeros
            for g in range(0, 32, 2):
                gm0 = gmax_v[pl.ds((b * 32 + g) * L, L)]
                gm1 = gmax_v[pl.ds((b * 32 + g + 1) * L, L)]
                bits0 = bits0 | jnp.where(gm0 > t0, _bit(g), jnp.int32(0))
                bits1 = bits1 | jnp.where(gm1 > t0, _bit(g + 1), jnp.int32(0))
            bm = _bf_or(bits0 | bits1)[0]

            @pl.when(bm != 0)
            def _():
                for g in range(32):
                    @pl.when((bm & _bit(g)) != 0)
                    def _(g=g):
                        n = nq_r[0]
                        glist_r[n] = jnp.int32(b * 32 + g)
                        nq_r[0] = n + 1

            return carry

        lax.fori_loop(0, GROUPS // 32, pass_b1, jnp.int32(0))
        nqual = nq_r[0]

        off_r[0] = jnp.int32(0)

        def pass_b2(j, carry):
            g = glist_r[j]
            base = dbase + g * (VPG * L)
            cbits0 = izeros
            cbits1 = izeros
            for k in range(0, VPG, 2):
                v0 = data_v[pl.ds(base + k * L, L)]
                v1 = data_v[pl.ds(base + (k + 1) * L, L)]
                cbits0 = cbits0 | jnp.where(v0 > t0, jnp.int32(1 << k),
                                            jnp.int32(0))
                cbits1 = cbits1 | jnp.where(v1 > t0, jnp.int32(1 << (k + 1)),
                                            jnp.int32(0))
            cm = _bf_or(cbits0 | cbits1)[0]
            for k in range(VPG):
                @pl.when((cm & (1 << k)) != 0)
                def _(k=k):
                    v = data_v[pl.ds(base + k * L, L)]
                    off = off_r[0]
                    cand_v[pl.ds(off * L, L)] = jnp.where(v > t0, v, t0)
                    off_r[0] = off + 1
            return carry

        lax.fori_loop(0, nqual, pass_b2, jnp.int32(0))
        ncand = off_r[0]
        pad = zeros + t0
        cand_v[pl.ds(ncand * L, L)] = pad
        cand_v[pl.ds(ncand * L + L, L)] = pad
        cand_v[pl.ds(ncand * L + 2 * L, L)] = pad
        nc2 = (ncand + 3) >> 2

        def stats_body(t):
            def body(i, c):
                cv, sv = c
                for u in range(4):
                    v = cand_v[pl.ds(i * (4 * L) + u * L, L)]
                    m = v > t
                    cv = cv + jnp.where(m, 1.0, 0.0)
                    sv = sv + jnp.where(m, v, 0.0)
                return cv, sv
            return body

        c0, s0 = lax.fori_loop(0, nc2, stats_body(t0), (zeros, zeros))
        t1 = ((_bf_sum(s0) - 1.0) / _bf_sum(c0))[0]
        c1, s1 = lax.fori_loop(0, nc2, stats_body(t1), (zeros, zeros))
        k1v = _bf_sum(c1)
        t2 = jnp.maximum(t1, ((_bf_sum(s1) - 1.0) / k1v)[0])
        n_iter = k1v[0].astype(jnp.int32) + 2

        def newton(_, st):
            t, done = st
            nc_eff = jnp.where(done, 0, nc2)
            cv, sv = lax.fori_loop(0, nc_eff, stats_body(t), (zeros, zeros))
            t3 = jnp.maximum(t, ((_bf_sum(sv) - 1.0) / _bf_sum(cv))[0])
            return t3, done | (t3 == t)

        tau, _ = lax.fori_loop(0, n_iter, newton, (t2, t2 == t1))

        def fin(i, c):
            cv, qv = c
            for u in range(4):
                v = cand_v[pl.ds(i * (4 * L) + u * L, L)]
                m = v > tau
                cv = cv + jnp.where(m, 1.0, 0.0)
                qv = qv + jnp.where(m, v * v, 0.0)
            return cv, qv

        cv, qv = lax.fori_loop(0, nc2, fin, (zeros, zeros))
        m_val = (0.5 * (_bf_sum(qv) - _bf_sum(cv) * tau * tau))[0]
        ms_vec = jnp.where(lax.iota(jnp.int32, L) == r, m_val, ms_vec)

    ms_v[...] = ms_vec
    pltpu.sync_copy(ms_v, out_hbm.at[pl.ds(wid * L, L)])


@jax.jit
def kernel(X):
    mesh = plsc.VectorSubcoreMesh(core_axis_name="c", subcore_axis_name="s")
    run = functools.partial(
        pl.kernel,
        mesh=mesh,
        out_type=jax.ShapeDtypeStruct((NW * L,), jnp.float32),
        compiler_params=pltpu.CompilerParams(use_tc_tiling_on_sc=True),
        scratch_types=[
            pltpu.VMEM((2 * N,), jnp.float32),
            pltpu.VMEM((GROUPS * L,), jnp.float32),
            pltpu.VMEM((N,), jnp.float32),
            pltpu.VMEM((L,), jnp.float32),
            pltpu.SMEM((1,), jnp.int32),
            pltpu.SMEM((1,), jnp.int32),
            pltpu.SMEM((GROUPS,), jnp.int32),
            pltpu.SemaphoreType.DMA,
            pltpu.SemaphoreType.DMA,
        ],
    )(_sc_kernel)
    out = run(X)
    return out.reshape(NW, L)[:, :ROWS_PER_W].reshape(B)


# speedup vs baseline: 1.0182x; 1.0182x over previous
import functools

import jax
import jax.numpy as jnp
from jax import lax
from jax.experimental import pallas as pl
from jax.experimental.pallas import tpu as pltpu
from jax.experimental.pallas import tpu_sc as plsc

L = 16
N = 32768
B = 128
NC = 2
NS = 16
NW = NC * NS
ROWS_PER_W = B // NW
VPG = 16
GROUPS = N // (VPG * L)
NBATCH = GROUPS // 16


def _bf_max(v):
    for sh in (1, 2, 4, 8):
        v = jnp.maximum(v, v[lax.iota(jnp.int32, L) ^ sh])
    return v


def _bf_sum(v):
    for sh in (1, 2, 4, 8):
        v = v + v[lax.iota(jnp.int32, L) ^ sh]
    return v


def _bf_or(v):
    for sh in (1, 2, 4, 8):
        v = v | v[lax.iota(jnp.int32, L) ^ sh]
    return v


def _bit(g):
    return jnp.int32(-(2 ** 31)) if g == 31 else jnp.int32(1 << g)


def _sc_kernel(x_hbm, out_hbm, data_v, gmax_v, cand_v, ms_v, off_r, nq_r,
               glist_r, sem_a, sem_b):
    wid = lax.axis_index("s") * NC + lax.axis_index("c")
    zeros = jnp.zeros((L,), jnp.float32)
    izeros = jnp.zeros((L,), jnp.int32)
    ms_vec = zeros
    sems = (sem_a, sem_b)

    copies = [None, None]
    copies[0] = pltpu.async_copy(
        x_hbm.at[wid * ROWS_PER_W], data_v.at[pl.ds(0, N)], sems[0])

    for r in range(ROWS_PER_W):
        row = wid * ROWS_PER_W + r
        dbase = (r % 2) * N
        copies[r % 2].wait()
        if r + 1 < ROWS_PER_W:
            copies[(r + 1) % 2] = pltpu.async_copy(
                x_hbm.at[row + 1],
                data_v.at[pl.ds(((r + 1) % 2) * N, N)], sems[(r + 1) % 2])

        def pass_a(g2, rowvmax):
            for half in range(2):
                base = dbase + (g2 * 2 + half) * (VPG * L)
                t0_ = data_v[pl.ds(base, L)]
                t1_ = data_v[pl.ds(base + L, L)]
                for k in range(2, VPG, 2):
                    t0_ = jnp.maximum(t0_, data_v[pl.ds(base + k * L, L)])
                    t1_ = jnp.maximum(
                        t1_, data_v[pl.ds(base + (k + 1) * L, L)])
                gm = jnp.maximum(t0_, t1_)
                gmax_v[pl.ds((g2 * 2 + half) * L, L)] = gm
                rowvmax = jnp.maximum(rowvmax, gm)
            return rowvmax

        rowvmax = lax.fori_loop(
            0, GROUPS // 2, pass_a, jnp.full((L,), -1e30, jnp.float32))
        rowmax = _bf_max(rowvmax)[0]
        t0 = rowmax - 1.0

        nq_r[0] = jnp.int32(0)

        def pass_b1(b, carry):
            bits = izeros
            for g in range(32):
                gm = gmax_v[pl.ds((b * 32 + g) * L, L)]
                bits = bits | jnp.where(gm > t0, _bit(g), jnp.int32(0))
            bm = _bf_or(bits)[0]

            @pl.when(bm != 0)
            def _():
                for g in range(32):
                    @pl.when((bm & _bit(g)) != 0)
                    def _(g=g):
                        n = nq_r[0]
                        glist_r[n] = jnp.int32(b * 32 + g)
                        nq_r[0] = n + 1

            return carry

        lax.fori_loop(0, GROUPS // 32, pass_b1, jnp.int32(0))
        nqual = nq_r[0]

        off_r[0] = jnp.int32(0)

        def pass_b2(j, carry):
            g = glist_r[j]
            base = dbase + g * (VPG * L)
            cbits = izeros
            for k in range(VPG):
                v = data_v[pl.ds(base + k * L, L)]
                cbits = cbits | jnp.where(v > t0, jnp.int32(1 << k),
                                          jnp.int32(0))
            cm = _bf_or(cbits)[0]
            for k in range(VPG):
                @pl.when((cm & (1 << k)) != 0)
                def _(k=k):
                    v = data_v[pl.ds(base + k * L, L)]
                    off = off_r[0]
                    cand_v[pl.ds(off * L, L)] = jnp.where(v > t0, v, t0)
                    off_r[0] = off + 1
            return carry

        lax.fori_loop(0, nqual, pass_b2, jnp.int32(0))
        ncand = off_r[0]
        pad = zeros + t0
        cand_v[pl.ds(ncand * L, L)] = pad
        cand_v[pl.ds(ncand * L + L, L)] = pad
        cand_v[pl.ds(ncand * L + 2 * L, L)] = pad
        nc2 = (ncand + 3) >> 2

        def stats_body(t):
            def body(i, c):
                cv, sv = c
                for u in range(4):
                    v = cand_v[pl.ds(i * (4 * L) + u * L, L)]
                    m = v > t
                    cv = cv + jnp.where(m, 1.0, 0.0)
                    sv = sv + jnp.where(m, v, 0.0)
                return cv, sv
            return body

        c0, s0 = lax.fori_loop(0, nc2, stats_body(t0), (zeros, zeros))
        t1 = ((_bf_sum(s0) - 1.0) / _bf_sum(c0))[0]
        c1, s1 = lax.fori_loop(0, nc2, stats_body(t1), (zeros, zeros))
        k1v = _bf_sum(c1)
        t2 = jnp.maximum(t1, ((_bf_sum(s1) - 1.0) / k1v)[0])
        n_iter = k1v[0].astype(jnp.int32) + 2

        def newton(_, st):
            t, done = st
            nc_eff = jnp.where(done, 0, nc2)
            cv, sv = lax.fori_loop(0, nc_eff, stats_body(t), (zeros, zeros))
            t3 = jnp.maximum(t, ((_bf_sum(sv) - 1.0) / _bf_sum(cv))[0])
            return t3, done | (t3 == t)

        tau, _ = lax.fori_loop(0, n_iter, newton, (t2, t2 == t1))

        def fin(i, c):
            cv, qv = c
            for u in range(4):
                v = cand_v[pl.ds(i * (4 * L) + u * L, L)]
                m = v > tau
                cv = cv + jnp.where(m, 1.0, 0.0)
                qv = qv + jnp.where(m, v * v, 0.0)
            return cv, qv

        cv, qv = lax.fori_loop(0, nc2, fin, (zeros, zeros))
        m_val = (0.5 * (_bf_sum(qv) - _bf_sum(cv) * tau * tau))[0]
        ms_vec = jnp.where(lax.iota(jnp.int32, L) == r, m_val, ms_vec)

    ms_v[...] = ms_vec
    pltpu.sync_copy(ms_v, out_hbm.at[pl.ds(wid * L, L)])


@jax.jit
def kernel(X):
    mesh = plsc.VectorSubcoreMesh(core_axis_name="c", subcore_axis_name="s")
    run = functools.partial(
        pl.kernel,
        mesh=mesh,
        out_type=jax.ShapeDtypeStruct((NW * L,), jnp.float32),
        compiler_params=pltpu.CompilerParams(use_tc_tiling_on_sc=True),
        scratch_types=[
            pltpu.VMEM((2 * N,), jnp.float32),
            pltpu.VMEM((GROUPS * L,), jnp.float32),
            pltpu.VMEM((N,), jnp.float32),
            pltpu.VMEM((L,), jnp.float32),
            pltpu.SMEM((1,), jnp.int32),
            pltpu.SMEM((1,), jnp.int32),
            pltpu.SMEM((GROUPS,), jnp.int32),
            pltpu.SemaphoreType.DMA,
            pltpu.SemaphoreType.DMA,
        ],
    )(_sc_kernel)
    out = run(X)
    return out.reshape(NW, L)[:, :ROWS_PER_W].reshape(B)
